# Initial kernel scaffold; baseline (speedup 1.0000x reference)
import jax
import jax.numpy as jnp
from jax.experimental import pallas as pl


def kernel(go_edge_index, back_edge_index, user_emb, item_emb, W1_go, b1_go, W1_back, b1_back, W2_go, b2_go, W2_back, b2_back, g1u, be1u, g1i, be1i, g2u, be2u, g2i, be2i):
    raise NotImplementedError("write your pallas kernel here")


# speedup vs baseline: 3.6072x; 3.6072x over previous
import functools

import jax
import jax.numpy as jnp
from jax import lax
from jax.experimental import pallas as pl
from jax.experimental.pallas import tpu as pltpu
from jax.experimental.pallas import tpu_sc as plsc

_N = 10000
_E = 320000
_D = 128
_NS = 16
_NP = 10240
_ZP = _NP // _NS
_CHUNK = 80
_EPT = _E // _NS
_NCH = _EPT // _CHUNK

@functools.cache
def _mesh():
    return plsc.VectorSubcoreMesh(core_axis_name="c", subcore_axis_name="s",
                                  num_cores=2, num_subcores=_NS)


def _scatter_body(xg, xb, sg, dg, sb, db, zeros, og, ob,
                  acc, idx_s, idx_d, rows, sem):
    c = lax.axis_index("c")
    s = lax.axis_index("s")

    pltpu.sync_copy(zeros.at[pl.ds(s * _ZP, _ZP)], acc.at[pl.ds(s * _ZP, _ZP)])
    plsc.subcore_barrier()

    def run(table, srcidx, dstidx, out):
        def step(i, carry):
            base = s * _EPT + i * _CHUNK
            pltpu.sync_copy(srcidx.at[pl.ds(base, _CHUNK)], idx_s)
            pltpu.sync_copy(dstidx.at[pl.ds(base, _CHUNK)], idx_d)
            pltpu.async_copy(table.at[idx_s], rows, sem).wait()
            pltpu.sync_copy(rows, acc.at[idx_d], add=True)
            return carry
        lax.fori_loop(0, _NCH, step, 0)
        plsc.subcore_barrier()
        pltpu.sync_copy(acc.at[pl.ds(s * _ZP, _ZP)], out.at[pl.ds(s * _ZP, _ZP)])

    pl.when(c == 0)(lambda: run(xg, sg, dg, og))
    pl.when(c == 1)(lambda: run(xb, sb, db, ob))


@functools.cache
def _sc_scatter():
    return pl.kernel(
        _scatter_body,
        out_type=(jax.ShapeDtypeStruct((_NP, _D), jnp.float32),
                  jax.ShapeDtypeStruct((_NP, _D), jnp.float32)),
        mesh=_mesh(),
        scratch_types=[
            pltpu.VMEM_SHARED((_NP, _D), jnp.float32),
            pltpu.VMEM((_CHUNK,), jnp.int32),
            pltpu.VMEM((_CHUNK,), jnp.int32),
            pltpu.VMEM((_CHUNK, _D), jnp.float32),
            pltpu.SemaphoreType.DMA,
        ],
    )


_DW = 128


def _degree_body(sg, dg, sb, db, zeros16, ones16, o_sg, o_dg, o_sb, o_db,
                 acc, onebuf, idxbuf):
    c = lax.axis_index("c")
    s = lax.axis_index("s")

    pltpu.sync_copy(ones16, onebuf)

    def phase(idxarr, out):
        pltpu.sync_copy(zeros16.at[pl.ds(s * _ZP, _ZP)],
                        acc.at[pl.ds(s * _ZP, _ZP)])
        plsc.subcore_barrier()

        def step(i, carry):
            base = s * _EPT + i * _CHUNK
            pltpu.sync_copy(idxarr.at[pl.ds(base, _CHUNK)], idxbuf)
            pltpu.sync_copy(onebuf, acc.at[idxbuf], add=True)
            return carry
        lax.fori_loop(0, _NCH, step, 0)
        plsc.subcore_barrier()
        pltpu.sync_copy(acc.at[pl.ds(s * _ZP, _ZP)], out.at[pl.ds(s * _ZP, _ZP)])
        plsc.subcore_barrier()

    def rel(a, b, oa, ob_):
        phase(a, oa)
        phase(b, ob_)

    pl.when(c == 0)(lambda: rel(sg, dg, o_sg, o_dg))
    pl.when(c == 1)(lambda: rel(sb, db, o_sb, o_db))


@functools.cache
def _sc_degrees():
    return pl.kernel(
        _degree_body,
        out_type=tuple(jax.ShapeDtypeStruct((_NP, _DW), jnp.float32)
                       for _ in range(4)),
        mesh=_mesh(),
        scratch_types=[
            pltpu.VMEM_SHARED((_NP, _DW), jnp.float32),
            pltpu.VMEM((_CHUNK, _DW), jnp.float32),
            pltpu.VMEM((_CHUNK,), jnp.int32),
        ],
    )


def _rsqrt_body(a, b, c, d, oa, ob, oc, od):
    oa[...] = lax.rsqrt(jnp.maximum(a[...], 1.0))
    ob[...] = lax.rsqrt(jnp.maximum(b[...], 1.0))
    oc[...] = lax.rsqrt(jnp.maximum(c[...], 1.0))
    od[...] = lax.rsqrt(jnp.maximum(d[...], 1.0))


def _tc_rsqrt(a, b, c, d):
    shp = jax.ShapeDtypeStruct(a.shape, jnp.float32)
    return pl.pallas_call(_rsqrt_body, out_shape=(shp, shp, shp, shp))(a, b, c, d)


def _scale_body(power, x1, s1, x2, s2, o1, o2):
    f1 = s1[...] ** power if power != 1 else s1[...]
    f2 = s2[...] ** power if power != 1 else s2[...]
    o1[...] = x1[...] * f1
    o2[...] = x2[...] * f2


def _tc_scale2(x1, s1, x2, s2, power):
    shp = jax.ShapeDtypeStruct((_N, _D), jnp.float32)
    return pl.pallas_call(functools.partial(_scale_body, power),
                          out_shape=(shp, shp))(x1, s1, x2, s2)


def _dense_body(with_pre, xd, a1, a2, ndv, w, b, g, be, nsn, *outs):
    h1 = a1[...] * ndv[...]
    h2 = a2[...] * ndv[...]
    wv = w[...]
    z = (jnp.dot(xd[...], wv[:_D], preferred_element_type=jnp.float32)
         + jnp.dot(h1, wv[_D:2 * _D], preferred_element_type=jnp.float32)
         + jnp.dot(h2, wv[2 * _D:], preferred_element_type=jnp.float32)
         + b[...])
    m = jnp.mean(z, axis=0, keepdims=True)
    v = jnp.mean((z - m) * (z - m), axis=0, keepdims=True)
    zn = g[...] * (z - m) * lax.rsqrt(v + 1e-5) + be[...]
    h = zn * jax.nn.sigmoid(zn)
    outs[0][...] = h
    if with_pre:
        outs[1][...] = h * nsn[...]


def _tc_dense(xd, a1, a2, ndv, w, b, g, be, nsn=None):
    shp = jax.ShapeDtypeStruct((_N, _D), jnp.float32)
    with_pre = nsn is not None
    out_shape = (shp, shp) if with_pre else (shp,)
    if nsn is None:
        nsn = jnp.zeros((_N, 1), jnp.float32)
    res = pl.pallas_call(functools.partial(_dense_body, with_pre),
                         out_shape=out_shape)(
        xd, a1, a2, ndv, w, b.reshape(1, _D), g.reshape(1, _D),
        be.reshape(1, _D), nsn)
    return res if with_pre else (res[0], None)


def kernel(go_edge_index, back_edge_index, user_emb, item_emb,
           W1_go, b1_go, W1_back, b1_back, W2_go, b2_go, W2_back, b2_back,
           g1u, be1u, g1i, be1i, g2u, be2u, g2i, be2i):
    gs, gd = go_edge_index[0], go_edge_index[1]
    bs, bd = back_edge_index[0], back_edge_index[1]
    zeros = jnp.zeros((_NP, _D), jnp.float32)
    ones16 = jnp.ones((_CHUNK, _DW), jnp.float32)

    deg = _sc_degrees()(gs, gd, bs, bd, zeros, ones16)
    ns_go, nd_go, ns_bk, nd_bk = (
        x[:_N, :1] for x in _tc_rsqrt(*deg))

    def layer(xsg, xsb, x_item, x_user, Wg, bg, Wb, bb, gi, bei, gu, beu,
              pre_next):
        a1g, a1b = _sc_scatter()(xsg, xsb, gs, gd, bs, bd, zeros)
        a1g, a1b = a1g[:_N], a1b[:_N]
        yg, yb = _tc_scale2(a1g, nd_go, a1b, nd_bk, 2)
        rg, rb = _sc_scatter()(yg, yb, gd, gs, bd, bs, zeros)
        zg, zb = _tc_scale2(rg[:_N], ns_go, rb[:_N], ns_bk, 2)
        a2g, a2b = _sc_scatter()(zg, zb, gs, gd, bs, bd, zeros)
        a2g, a2b = a2g[:_N], a2b[:_N]
        h_item, item_pre = _tc_dense(x_item, a1g, a2g, nd_go, Wg, bg, gi, bei,
                                     nsn=ns_bk if pre_next else None)
        h_user, user_pre = _tc_dense(x_user, a1b, a2b, nd_bk, Wb, bb, gu, beu,
                                     nsn=ns_go if pre_next else None)
        return h_item, h_user, item_pre, user_pre

    xsg1, xsb1 = _tc_scale2(user_emb, ns_go, item_emb, ns_bk, 1)
    h_item1, h_user1, item_pre, user_pre = layer(
        xsg1, xsb1, item_emb, user_emb,
        W1_go, b1_go, W1_back, b1_back, g1i, be1i, g1u, be1u, pre_next=True)

    h_item2, h_user2, _, _ = layer(
        user_pre, item_pre, h_item1, h_user1,
        W2_go, b2_go, W2_back, b2_back, g2i, be2i, g2u, be2u, pre_next=False)

    return (h_user2, h_item2)


# speedup vs baseline: 3.8906x; 1.0786x over previous
import functools

import jax
import jax.numpy as jnp
from jax import lax
from jax.experimental import pallas as pl
from jax.experimental.pallas import tpu as pltpu
from jax.experimental.pallas import tpu_sc as plsc

_N = 10000
_E = 320000
_D = 128
_NS = 16
_NP = 10240
_ZP = _NP // _NS
_JUNK = _NP - 8
_CHUNK = 128
_EP = 327680
_NCHT = _EP // _CHUNK // _NS
_BLK = 8
_NBLK = _NCHT // _BLK

@functools.cache
def _mesh():
    return plsc.VectorSubcoreMesh(core_axis_name="c", subcore_axis_name="s",
                                  num_cores=2, num_subcores=_NS)


def _scatter_body(xg, xb, sg, dg, sb, db, zeros, og, ob,
                  acc, idx_s, idx_d, rows0, rows1,
                  sg0, sg1, ss0, ss1):
    c = lax.axis_index("c")
    s = lax.axis_index("s")
    rows = (rows0, rows1)
    gsem = (sg0, sg1)
    ssem = (ss0, ss1)

    pltpu.sync_copy(zeros.at[pl.ds(s * _ZP, _ZP)], acc.at[pl.ds(s * _ZP, _ZP)])
    plsc.subcore_barrier()

    def run(table, srcidx, dstidx, out):
        def blk(bi, carry):
            rowbase = s * _NCHT + bi * _BLK
            pltpu.sync_copy(srcidx.at[pl.ds(rowbase, _BLK)], idx_s)
            pltpu.sync_copy(dstidx.at[pl.ds(rowbase, _BLK)], idx_d)
            dg_ = [None] * _BLK
            ds_ = [None] * _BLK
            dg_[0] = pltpu.async_copy(table.at[idx_s.at[0]], rows0, sg0)
            for j in range(_BLK):
                b = j % 2
                if j + 1 < _BLK:
                    if j >= 1:
                        ds_[j - 1].wait()
                    dg_[j + 1] = pltpu.async_copy(
                        table.at[idx_s.at[j + 1]], rows[(j + 1) % 2],
                        gsem[(j + 1) % 2])
                dg_[j].wait()
                ds_[j] = pltpu.async_copy(rows[b], acc.at[idx_d.at[j]],
                                          ssem[b], add=True)
            ds_[_BLK - 2].wait()
            ds_[_BLK - 1].wait()
            return carry
        lax.fori_loop(0, _NBLK, blk, 0)
        plsc.subcore_barrier()
        pltpu.sync_copy(acc.at[pl.ds(s * _ZP, _ZP)], out.at[pl.ds(s * _ZP, _ZP)])

    pl.when(c == 0)(lambda: run(xg, sg, dg, og))
    pl.when(c == 1)(lambda: run(xb, sb, db, ob))


@functools.cache
def _sc_scatter():
    return pl.kernel(
        _scatter_body,
        out_type=(jax.ShapeDtypeStruct((_NP, _D), jnp.float32),
                  jax.ShapeDtypeStruct((_NP, _D), jnp.float32)),
        mesh=_mesh(),
        scratch_types=[
            pltpu.VMEM_SHARED((_NP, _D), jnp.float32),
            pltpu.VMEM((_BLK, _CHUNK), jnp.int32),
            pltpu.VMEM((_BLK, _CHUNK), jnp.int32),
            pltpu.VMEM((_CHUNK, _D), jnp.float32),
            pltpu.VMEM((_CHUNK, _D), jnp.float32),
            pltpu.SemaphoreType.DMA,
            pltpu.SemaphoreType.DMA,
            pltpu.SemaphoreType.DMA,
            pltpu.SemaphoreType.DMA,
        ],
    )


_DW = 128


def _degree_body(sg, dg, sb, db, zeros16, ones16, o_sg, o_dg, o_sb, o_db,
                 acc, onebuf, idxbuf, sem):
    c = lax.axis_index("c")
    s = lax.axis_index("s")

    pltpu.sync_copy(ones16, onebuf)

    def phase(idxarr, out):
        pltpu.sync_copy(zeros16.at[pl.ds(s * _ZP, _ZP)],
                        acc.at[pl.ds(s * _ZP, _ZP)])
        plsc.subcore_barrier()

        def blk(bi, carry):
            rowbase = s * _NCHT + bi * _BLK
            pltpu.sync_copy(idxarr.at[pl.ds(rowbase, _BLK)], idxbuf)
            ds_ = [pltpu.async_copy(onebuf, acc.at[idxbuf.at[j]], sem,
                                    add=True)
                   for j in range(_BLK)]
            for d in ds_:
                d.wait()
            return carry
        lax.fori_loop(0, _NBLK, blk, 0)
        plsc.subcore_barrier()
        pltpu.sync_copy(acc.at[pl.ds(s * _ZP, _ZP)], out.at[pl.ds(s * _ZP, _ZP)])
        plsc.subcore_barrier()

    def rel(a, b, oa, ob_):
        phase(a, oa)
        phase(b, ob_)

    pl.when(c == 0)(lambda: rel(sg, dg, o_sg, o_dg))
    pl.when(c == 1)(lambda: rel(sb, db, o_sb, o_db))


@functools.cache
def _sc_degrees():
    return pl.kernel(
        _degree_body,
        out_type=tuple(jax.ShapeDtypeStruct((_NP, _DW), jnp.float32)
                       for _ in range(4)),
        mesh=_mesh(),
        scratch_types=[
            pltpu.VMEM_SHARED((_NP, _DW), jnp.float32),
            pltpu.VMEM((_CHUNK, _DW), jnp.float32),
            pltpu.VMEM((_BLK, _CHUNK), jnp.int32),
            pltpu.SemaphoreType.DMA,
        ],
    )


def _rsqrt_body(a, b, c, d, oa, ob, oc, od):
    oa[...] = lax.rsqrt(jnp.maximum(a[...], 1.0))
    ob[...] = lax.rsqrt(jnp.maximum(b[...], 1.0))
    oc[...] = lax.rsqrt(jnp.maximum(c[...], 1.0))
    od[...] = lax.rsqrt(jnp.maximum(d[...], 1.0))


def _tc_rsqrt(a, b, c, d):
    shp = jax.ShapeDtypeStruct(a.shape, jnp.float32)
    return pl.pallas_call(_rsqrt_body, out_shape=(shp, shp, shp, shp))(a, b, c, d)


def _scale_body(power, x1, s1, x2, s2, o1, o2):
    f1 = s1[...] ** power if power != 1 else s1[...]
    f2 = s2[...] ** power if power != 1 else s2[...]
    o1[...] = x1[...] * f1
    o2[...] = x2[...] * f2


def _tc_scale2(x1, s1, x2, s2, power):
    shp = jax.ShapeDtypeStruct((_NP, _D), jnp.float32)
    return pl.pallas_call(functools.partial(_scale_body, power),
                          out_shape=(shp, shp))(x1, s1, x2, s2)


def _dense_body(with_pre, xd, a1, a2, ndv, w, b, g, be, nsn, *outs):
    h1 = a1[...] * ndv[...]
    h2 = a2[...] * ndv[...]
    wv = w[...]
    z = (jnp.dot(xd[...], wv[:_D], preferred_element_type=jnp.float32)
         + jnp.dot(h1, wv[_D:2 * _D], preferred_element_type=jnp.float32)
         + jnp.dot(h2, wv[2 * _D:], preferred_element_type=jnp.float32)
         + b[...])
    m = jnp.mean(z, axis=0, keepdims=True)
    v = jnp.mean((z - m) * (z - m), axis=0, keepdims=True)
    zn = g[...] * (z - m) * lax.rsqrt(v + 1e-5) + be[...]
    h = zn * jax.nn.sigmoid(zn)
    outs[0][...] = h
    if with_pre:
        outs[1][...] = h * nsn[...]


def _tc_dense(xd, a1, a2, ndv, w, b, g, be, nsn=None):
    shp = jax.ShapeDtypeStruct((_N, _D), jnp.float32)
    with_pre = nsn is not None
    out_shape = (shp, shp) if with_pre else (shp,)
    if nsn is None:
        nsn = jnp.zeros((_N, 1), jnp.float32)
    res = pl.pallas_call(functools.partial(_dense_body, with_pre),
                         out_shape=out_shape)(
        xd, a1, a2, ndv, w, b.reshape(1, _D), g.reshape(1, _D),
        be.reshape(1, _D), nsn)
    return res if with_pre else (res[0], None)


def kernel(go_edge_index, back_edge_index, user_emb, item_emb,
           W1_go, b1_go, W1_back, b1_back, W2_go, b2_go, W2_back, b2_back,
           g1u, be1u, g1i, be1i, g2u, be2u, g2i, be2i):
    zeros = jnp.zeros((_NP, _D), jnp.float32)
    ones2 = jnp.ones((_CHUNK, _DW), jnp.float32)
    padi = jnp.full((_EP - _E,), _JUNK, jnp.int32)
    gs, gd, bs, bd = (
        jnp.concatenate([idx.astype(jnp.int32), padi])
           .reshape(_EP // _CHUNK, _CHUNK)
        for idx in (go_edge_index[0], go_edge_index[1],
                    back_edge_index[0], back_edge_index[1]))
    rowpad = jnp.zeros((_NP - _N, _D), jnp.float32)
    user_pad = jnp.concatenate([user_emb, rowpad])
    item_pad = jnp.concatenate([item_emb, rowpad])

    deg = _sc_degrees()(gs, gd, bs, bd, zeros, ones2)
    ns_goP, nd_goP, ns_bkP, nd_bkP = (x[:, :1] for x in _tc_rsqrt(*deg))
    ns_go, nd_go, ns_bk, nd_bk = (x[:_N] for x in
                                  (ns_goP, nd_goP, ns_bkP, nd_bkP))

    def layer(xsg, xsb, x_item, x_user, Wg, bg, Wb, bb, gi, bei, gu, beu,
              pre_next):
        a1g, a1b = _sc_scatter()(xsg, xsb, gs, gd, bs, bd, zeros)
        yg, yb = _tc_scale2(a1g, nd_goP, a1b, nd_bkP, 2)
        rg, rb = _sc_scatter()(yg, yb, gd, gs, bd, bs, zeros)
        zg, zb = _tc_scale2(rg, ns_goP, rb, ns_bkP, 2)
        a2g, a2b = _sc_scatter()(zg, zb, gs, gd, bs, bd, zeros)
        h_item, item_pre = _tc_dense(x_item, a1g[:_N], a2g[:_N], nd_go,
                                     Wg, bg, gi, bei,
                                     nsn=ns_bk if pre_next else None)
        h_user, user_pre = _tc_dense(x_user, a1b[:_N], a2b[:_N], nd_bk,
                                     Wb, bb, gu, beu,
                                     nsn=ns_go if pre_next else None)
        return h_item, h_user, item_pre, user_pre

    xsg1, xsb1 = _tc_scale2(user_pad, ns_goP, item_pad, ns_bkP, 1)
    h_item1, h_user1, item_pre, user_pre = layer(
        xsg1, xsb1, item_emb, user_emb,
        W1_go, b1_go, W1_back, b1_back, g1i, be1i, g1u, be1u, pre_next=True)

    h_item2, h_user2, _, _ = layer(
        jnp.concatenate([user_pre, rowpad]), jnp.concatenate([item_pre, rowpad]),
        h_item1, h_user1,
        W2_go, b2_go, W2_back, b2_back, g2i, be2i, g2u, be2u, pre_next=False)

    return (h_user2, h_item2)


# speedup vs baseline: 3.8947x; 1.0011x over previous
import functools

import jax
import jax.numpy as jnp
from jax import lax
from jax.experimental import pallas as pl
from jax.experimental.pallas import tpu as pltpu
from jax.experimental.pallas import tpu_sc as plsc

_N = 10000
_E = 320000
_D = 128
_NS = 16
_NP = 10240
_ZP = _NP // _NS
_JUNK = _NP - 8
_CHUNK = 64
_EP = 327680
_NCHT = _EP // _CHUNK // _NS
_BLK = 16
_NBLK = _NCHT // _BLK
_NB = 4

@functools.cache
def _mesh():
    return plsc.VectorSubcoreMesh(core_axis_name="c", subcore_axis_name="s",
                                  num_cores=2, num_subcores=_NS)


def _scatter_body(xg, xb, sg, dg, sb, db, zeros, og, ob,
                  acc, idx_s, idx_d, *bufs):
    c = lax.axis_index("c")
    s = lax.axis_index("s")
    rows = bufs[:_NB]
    gsem = bufs[_NB:2 * _NB]
    ssem = bufs[2 * _NB:3 * _NB]

    pltpu.sync_copy(zeros.at[pl.ds(s * _ZP, _ZP)], acc.at[pl.ds(s * _ZP, _ZP)])
    plsc.subcore_barrier()

    def run(table, srcidx, dstidx, out):
        def blk(bi, carry):
            rowbase = s * _NCHT + bi * _BLK
            pltpu.sync_copy(srcidx.at[pl.ds(rowbase, _BLK)], idx_s)
            pltpu.sync_copy(dstidx.at[pl.ds(rowbase, _BLK)], idx_d)
            dg_ = [None] * _BLK
            ds_ = [None] * _BLK
            for p in range(_NB - 1):
                dg_[p] = pltpu.async_copy(table.at[idx_s.at[p]], rows[p],
                                          gsem[p])
            for j in range(_BLK):
                nj = j + _NB - 1
                if nj < _BLK:
                    if nj - _NB >= 0:
                        ds_[nj - _NB].wait()
                    dg_[nj] = pltpu.async_copy(
                        table.at[idx_s.at[nj]], rows[nj % _NB],
                        gsem[nj % _NB])
                dg_[j].wait()
                ds_[j] = pltpu.async_copy(rows[j % _NB], acc.at[idx_d.at[j]],
                                          ssem[j % _NB], add=True)
            for j in range(max(0, _BLK - _NB), _BLK):
                ds_[j].wait()
            return carry
        lax.fori_loop(0, _NBLK, blk, 0)
        plsc.subcore_barrier()
        pltpu.sync_copy(acc.at[pl.ds(s * _ZP, _ZP)], out.at[pl.ds(s * _ZP, _ZP)])

    pl.when(c == 0)(lambda: run(xg, sg, dg, og))
    pl.when(c == 1)(lambda: run(xb, sb, db, ob))


@functools.cache
def _sc_scatter():
    return pl.kernel(
        _scatter_body,
        out_type=(jax.ShapeDtypeStruct((_NP, _D), jnp.float32),
                  jax.ShapeDtypeStruct((_NP, _D), jnp.float32)),
        mesh=_mesh(),
        scratch_types=[
            pltpu.VMEM_SHARED((_NP, _D), jnp.float32),
            pltpu.VMEM((_BLK, _CHUNK), jnp.int32),
            pltpu.VMEM((_BLK, _CHUNK), jnp.int32),
        ] + [pltpu.VMEM((_CHUNK, _D), jnp.float32) for _ in range(_NB)]
          + [pltpu.SemaphoreType.DMA for _ in range(2 * _NB)],
    )


_DW = 128


def _degree_body(sg, dg, sb, db, zeros16, ones16, o_sg, o_dg, o_sb, o_db,
                 acc, onebuf, idxbuf, sem):
    c = lax.axis_index("c")
    s = lax.axis_index("s")

    pltpu.sync_copy(ones16, onebuf)

    def phase(idxarr, out):
        pltpu.sync_copy(zeros16.at[pl.ds(s * _ZP, _ZP)],
                        acc.at[pl.ds(s * _ZP, _ZP)])
        plsc.subcore_barrier()

        def blk(bi, carry):
            rowbase = s * _NCHT + bi * _BLK
            pltpu.sync_copy(idxarr.at[pl.ds(rowbase, _BLK)], idxbuf)
            ds_ = [pltpu.async_copy(onebuf, acc.at[idxbuf.at[j]], sem,
                                    add=True)
                   for j in range(_BLK)]
            for d in ds_:
                d.wait()
            return carry
        lax.fori_loop(0, _NBLK, blk, 0)
        plsc.subcore_barrier()
        pltpu.sync_copy(acc.at[pl.ds(s * _ZP, _ZP)], out.at[pl.ds(s * _ZP, _ZP)])
        plsc.subcore_barrier()

    def rel(a, b, oa, ob_):
        phase(a, oa)
        phase(b, ob_)

    pl.when(c == 0)(lambda: rel(sg, dg, o_sg, o_dg))
    pl.when(c == 1)(lambda: rel(sb, db, o_sb, o_db))


@functools.cache
def _sc_degrees():
    return pl.kernel(
        _degree_body,
        out_type=tuple(jax.ShapeDtypeStruct((_NP, _DW), jnp.float32)
                       for _ in range(4)),
        mesh=_mesh(),
        scratch_types=[
            pltpu.VMEM_SHARED((_NP, _DW), jnp.float32),
            pltpu.VMEM((_CHUNK, _DW), jnp.float32),
            pltpu.VMEM((_BLK, _CHUNK), jnp.int32),
            pltpu.SemaphoreType.DMA,
        ],
    )


def _rsqrt_body(a, b, c, d, oa, ob, oc, od):
    oa[...] = lax.rsqrt(jnp.maximum(a[...], 1.0))
    ob[...] = lax.rsqrt(jnp.maximum(b[...], 1.0))
    oc[...] = lax.rsqrt(jnp.maximum(c[...], 1.0))
    od[...] = lax.rsqrt(jnp.maximum(d[...], 1.0))


def _tc_rsqrt(a, b, c, d):
    shp = jax.ShapeDtypeStruct(a.shape, jnp.float32)
    return pl.pallas_call(_rsqrt_body, out_shape=(shp, shp, shp, shp))(a, b, c, d)


def _scale_body(power, x1, s1, x2, s2, o1, o2):
    f1 = s1[...] ** power if power != 1 else s1[...]
    f2 = s2[...] ** power if power != 1 else s2[...]
    o1[...] = x1[...] * f1
    o2[...] = x2[...] * f2


def _tc_scale2(x1, s1, x2, s2, power):
    shp = jax.ShapeDtypeStruct((_NP, _D), jnp.float32)
    return pl.pallas_call(functools.partial(_scale_body, power),
                          out_shape=(shp, shp))(x1, s1, x2, s2)


def _dense_body(with_pre, xd, a1, a2, ndv, w, b, g, be, nsn, *outs):
    h1 = a1[...] * ndv[...]
    h2 = a2[...] * ndv[...]
    wv = w[...]
    z = (jnp.dot(xd[...], wv[:_D], preferred_element_type=jnp.float32)
         + jnp.dot(h1, wv[_D:2 * _D], preferred_element_type=jnp.float32)
         + jnp.dot(h2, wv[2 * _D:], preferred_element_type=jnp.float32)
         + b[...])
    m = jnp.mean(z, axis=0, keepdims=True)
    v = jnp.mean((z - m) * (z - m), axis=0, keepdims=True)
    zn = g[...] * (z - m) * lax.rsqrt(v + 1e-5) + be[...]
    h = zn * jax.nn.sigmoid(zn)
    outs[0][...] = h
    if with_pre:
        outs[1][...] = h * nsn[...]


def _tc_dense(xd, a1, a2, ndv, w, b, g, be, nsn=None):
    shp = jax.ShapeDtypeStruct((_N, _D), jnp.float32)
    with_pre = nsn is not None
    out_shape = (shp, shp) if with_pre else (shp,)
    if nsn is None:
        nsn = jnp.zeros((_N, 1), jnp.float32)
    res = pl.pallas_call(functools.partial(_dense_body, with_pre),
                         out_shape=out_shape)(
        xd, a1, a2, ndv, w, b.reshape(1, _D), g.reshape(1, _D),
        be.reshape(1, _D), nsn)
    return res if with_pre else (res[0], None)


def kernel(go_edge_index, back_edge_index, user_emb, item_emb,
           W1_go, b1_go, W1_back, b1_back, W2_go, b2_go, W2_back, b2_back,
           g1u, be1u, g1i, be1i, g2u, be2u, g2i, be2i):
    zeros = jnp.zeros((_NP, _D), jnp.float32)
    ones2 = jnp.ones((_CHUNK, _DW), jnp.float32)
    padi = jnp.full((_EP - _E,), _JUNK, jnp.int32)
    gs, gd, bs, bd = (
        jnp.concatenate([idx.astype(jnp.int32), padi])
           .reshape(_EP // _CHUNK, _CHUNK)
        for idx in (go_edge_index[0], go_edge_index[1],
                    back_edge_index[0], back_edge_index[1]))
    rowpad = jnp.zeros((_NP - _N, _D), jnp.float32)
    user_pad = jnp.concatenate([user_emb, rowpad])
    item_pad = jnp.concatenate([item_emb, rowpad])

    deg = _sc_degrees()(gs, gd, bs, bd, zeros, ones2)
    ns_goP, nd_goP, ns_bkP, nd_bkP = (x[:, :1] for x in _tc_rsqrt(*deg))
    ns_go, nd_go, ns_bk, nd_bk = (x[:_N] for x in
                                  (ns_goP, nd_goP, ns_bkP, nd_bkP))

    def layer(xsg, xsb, x_item, x_user, Wg, bg, Wb, bb, gi, bei, gu, beu,
              pre_next):
        a1g, a1b = _sc_scatter()(xsg, xsb, gs, gd, bs, bd, zeros)
        yg, yb = _tc_scale2(a1g, nd_goP, a1b, nd_bkP, 2)
        rg, rb = _sc_scatter()(yg, yb, gd, gs, bd, bs, zeros)
        zg, zb = _tc_scale2(rg, ns_goP, rb, ns_bkP, 2)
        a2g, a2b = _sc_scatter()(zg, zb, gs, gd, bs, bd, zeros)
        h_item, item_pre = _tc_dense(x_item, a1g[:_N], a2g[:_N], nd_go,
                                     Wg, bg, gi, bei,
                                     nsn=ns_bk if pre_next else None)
        h_user, user_pre = _tc_dense(x_user, a1b[:_N], a2b[:_N], nd_bk,
                                     Wb, bb, gu, beu,
                                     nsn=ns_go if pre_next else None)
        return h_item, h_user, item_pre, user_pre

    xsg1, xsb1 = _tc_scale2(user_pad, ns_goP, item_pad, ns_bkP, 1)
    h_item1, h_user1, item_pre, user_pre = layer(
        xsg1, xsb1, item_emb, user_emb,
        W1_go, b1_go, W1_back, b1_back, g1i, be1i, g1u, be1u, pre_next=True)

    h_item2, h_user2, _, _ = layer(
        jnp.concatenate([user_pre, rowpad]), jnp.concatenate([item_pre, rowpad]),
        h_item1, h_user1,
        W2_go, b2_go, W2_back, b2_back, g2i, be2i, g2u, be2u, pre_next=False)

    return (h_user2, h_item2)
